# baseline (speedup 1.0000x reference)
import functools

import jax
import jax.numpy as jnp
from jax import lax
from jax.experimental import pallas as pl
from jax.experimental.pallas import tpu as pltpu
from jax.experimental.pallas import tpu_sc as plsc

_B = 4096
_DIC = 100
_D = 64
_NE = 6
_DP = _DIC // 2
_NPAIR = _B * _DP
_NC = 2
_NS = 16
_NW = _NC * _NS
_PER_W = _NPAIR // _NW
_RPW = _B // _NW
_CH = 128
_NCH = _PER_W // _CH
_NB = 2


def _tab_body(nm2_ref, vt_ref, out_ref):
    nm2 = nm2_ref[...]
    zero = jnp.zeros((_D,), jnp.float32)
    for e0 in range(_NE + 1):
        left = vt_ref[e0] if e0 < _NE else zero
        for e1 in range(_NE + 1):
            right = vt_ref[e1] if e1 < _NE else zero
            out_ref[e0, e1] = nm2 + jnp.concatenate([left, right], axis=-1)


def _build_table(name_embedding, value_table):
    out = pl.pallas_call(
        _tab_body,
        out_shape=jax.ShapeDtypeStruct((_NE + 1, _NE + 1, _DP, 2 * _D), jnp.float32),
    )(name_embedding.reshape(_DP, 2 * _D), value_table)
    return out.reshape((_NE + 1) * (_NE + 1) * _DP, 2 * _D)


def _sc_body(ctab_h, p_h, dpl_h, out_h, p_v, dp_v, i_v, stage_v, sem_g, sem_s):
    wid = lax.axis_index("s") * _NC + lax.axis_index("c")
    base0 = wid * _PER_W
    pltpu.sync_copy(p_h.at[pl.ds(wid * _RPW * 128, _RPW * 128)], p_v)
    pltpu.sync_copy(dpl_h, dp_v)

    def idx_row(r, carry):
        base = r * 128
        for off in (0, 16, 32, 34):
            xc = p_v[pl.ds(base + off, 16)]
            yv = p_v[pl.ds(base + 64 + off, 16)]
            dp = dp_v[pl.ds(off, 16)]
            xe = xc & 7
            xo = xc >> 3
            d0 = dp * 2
            e0 = jnp.where(d0 == yv, _NE, xe)
            e1 = jnp.where(d0 + 1 == yv, _NE, xo)
            i_v[pl.ds(r * _DP + off, 16)] = (e0 * (_NE + 1) + e1) * _DP + dp
        return carry

    lax.fori_loop(0, _RPW, idx_row, 0)

    def start_gather(c, b):
        pltpu.async_copy(
            ctab_h.at[i_v.at[pl.ds(c * _CH, _CH)]], stage_v.at[b], sem_g)

    def wait_gather(c, b):
        pltpu.make_async_copy(
            ctab_h.at[i_v.at[pl.ds(c * _CH, _CH)]], stage_v.at[b], sem_g).wait()

    for b in range(_NB):
        start_gather(b, b)

    def outer(t, carry):
        c0 = t * _NB
        for b in range(_NB):
            c = c0 + b
            base = base0 + c * _CH
            wait_gather(c, b)
            pltpu.async_copy(stage_v.at[b], out_h.at[pl.ds(base, _CH)], sem_s)
            pltpu.make_async_copy(
                stage_v.at[b], out_h.at[pl.ds(base, _CH)], sem_s).wait()

            @pl.when(c + _NB < _NCH)
            def _():
                start_gather(c + _NB, b)
        return carry

    lax.fori_loop(0, _NCH // _NB, outer, 0)


def _sc_gather(ctab, packed, dpl):
    mesh = plsc.VectorSubcoreMesh(core_axis_name="c", subcore_axis_name="s")
    run = functools.partial(
        pl.kernel,
        out_type=jax.ShapeDtypeStruct((_NPAIR, 2 * _D), jnp.float32),
        mesh=mesh,
        scratch_types=[
            pltpu.VMEM((_RPW * 128,), jnp.int32),
            pltpu.VMEM((_DP,), jnp.int32),
            pltpu.VMEM((_PER_W,), jnp.int32),
            pltpu.VMEM((_NB, _CH, 2 * _D), jnp.float32),
            pltpu.SemaphoreType.DMA,
            pltpu.SemaphoreType.DMA,
        ],
    )(_sc_body)
    return run(ctab, packed, dpl)


def _pack_inputs(x, y):
    xc = x[:, 0::2] + 8 * x[:, 1::2]
    pad = jnp.zeros((_B, 64 - _DP), jnp.int32)
    yb = jnp.broadcast_to(y[:, None], (_B, 56))
    pad2 = jnp.zeros((_B, 8), jnp.int32)
    return jnp.concatenate([xc, pad, yb, pad2], axis=1).reshape(_B * 128)


@jax.jit
def kernel(x, y, name_embedding, value_table):
    x = x.astype(jnp.int32)
    y = y.astype(jnp.int32)
    ctab = _build_table(name_embedding, value_table)
    packed = _pack_inputs(x, y)
    dpl = jnp.arange(_DP, dtype=jnp.int32)
    mid = _sc_gather(ctab, packed, dpl)
    one = value_table[0, 0] * 0.0 + 1.0
    return mid.reshape(_B, _DIC, _D) * one


# speedup vs baseline: 1.1699x; 1.1699x over previous
import functools

import jax
import jax.numpy as jnp
from jax import lax
from jax.experimental import pallas as pl
from jax.experimental.pallas import tpu as pltpu
from jax.experimental.pallas import tpu_sc as plsc

_B = 4096
_DIC = 100
_D = 64
_NE = 6
_DP = _DIC // 2
_NPAIR = _B * _DP
_NC = 2
_NS = 16
_NW = _NC * _NS
_PER_W = _NPAIR // _NW
_RPW = _B // _NW
_CH = 128
_NCH = _PER_W // _CH
_NB = 2


def _tab_body(nm2_ref, vt_ref, out_ref):
    nm2 = nm2_ref[...]
    zero = jnp.zeros((_D,), jnp.float32)
    for e0 in range(_NE + 1):
        left = vt_ref[e0] if e0 < _NE else zero
        for e1 in range(_NE + 1):
            right = vt_ref[e1] if e1 < _NE else zero
            out_ref[e0, e1] = nm2 + jnp.concatenate([left, right], axis=-1)


def _build_table(name_embedding, value_table):
    out = pl.pallas_call(
        _tab_body,
        out_shape=jax.ShapeDtypeStruct((_NE + 1, _NE + 1, _DP, 2 * _D), jnp.float32),
    )(name_embedding.reshape(_DP, 2 * _D), value_table)
    return out.reshape((_NE + 1) * (_NE + 1) * _DP, 2 * _D)


def _sc_body(ctab_h, p_h, dpl_h, out_h, p_v, dp_v, i_v, stage_v, sem_g, sem_s):
    wid = lax.axis_index("s") * _NC + lax.axis_index("c")
    base0 = wid * _PER_W
    pltpu.sync_copy(p_h.at[pl.ds(wid * _RPW * 128, _RPW * 128)], p_v)
    pltpu.sync_copy(dpl_h, dp_v)

    def idx_row(r, carry):
        base = r * 128
        for off in (0, 16, 32, 34):
            xc = p_v[pl.ds(base + off, 16)]
            yv = p_v[pl.ds(base + 64 + off, 16)]
            dp = dp_v[pl.ds(off, 16)]
            xe = xc & 7
            xo = xc >> 3
            d0 = dp * 2
            e0 = jnp.where(d0 == yv, _NE, xe)
            e1 = jnp.where(d0 + 1 == yv, _NE, xo)
            i_v[pl.ds(r * _DP + off, 16)] = (e0 * (_NE + 1) + e1) * _DP + dp
        return carry

    lax.fori_loop(0, _RPW, idx_row, 0)

    def start_gather(c, b):
        pltpu.async_copy(
            ctab_h.at[i_v.at[pl.ds(c * _CH, _CH)]], stage_v.at[b], sem_g)

    def wait_gather(c, b):
        pltpu.make_async_copy(
            ctab_h.at[i_v.at[pl.ds(c * _CH, _CH)]], stage_v.at[b], sem_g).wait()

    for b in range(_NB):
        start_gather(b, b)

    def outer(t, carry):
        c0 = t * _NB
        for b in range(_NB):
            c = c0 + b
            base = base0 + c * _CH
            wait_gather(c, b)
            pltpu.async_copy(stage_v.at[b], out_h.at[pl.ds(base, _CH)], sem_s)
            pltpu.make_async_copy(
                stage_v.at[b], out_h.at[pl.ds(base, _CH)], sem_s).wait()

            @pl.when(c + _NB < _NCH)
            def _():
                start_gather(c + _NB, b)
        return carry

    lax.fori_loop(0, _NCH // _NB, outer, 0)


def _sc_gather(ctab, packed, dpl):
    mesh = plsc.VectorSubcoreMesh(core_axis_name="c", subcore_axis_name="s")
    run = functools.partial(
        pl.kernel,
        out_type=jax.ShapeDtypeStruct((_NPAIR, 2 * _D), jnp.float32),
        mesh=mesh,
        scratch_types=[
            pltpu.VMEM((_RPW * 128,), jnp.int32),
            pltpu.VMEM((_DP,), jnp.int32),
            pltpu.VMEM((_PER_W,), jnp.int32),
            pltpu.VMEM((_NB, _CH, 2 * _D), jnp.float32),
            pltpu.SemaphoreType.DMA,
            pltpu.SemaphoreType.DMA,
        ],
    )(_sc_body)
    return run(ctab, packed, dpl)


def _pack_inputs(x, y):
    xc = x[:, 0::2] + 8 * x[:, 1::2]
    pad = jnp.zeros((_B, 64 - _DP), jnp.int32)
    yb = jnp.broadcast_to(y[:, None], (_B, 56))
    pad2 = jnp.zeros((_B, 8), jnp.int32)
    return jnp.concatenate([xc, pad, yb, pad2], axis=1).reshape(_B * 128)


@jax.jit
def kernel(x, y, name_embedding, value_table):
    x = x.astype(jnp.int32)
    y = y.astype(jnp.int32)
    ctab = _build_table(name_embedding, value_table)
    packed = _pack_inputs(x, y)
    dpl = jnp.arange(_DP, dtype=jnp.int32)
    mid = _sc_gather(ctab, packed, dpl)
    return mid.reshape(_B, _DIC, _D)
